# baseline (speedup 1.0000x reference)
import functools

import jax
import jax.numpy as jnp
from jax import lax
from jax.experimental import pallas as pl
from jax.experimental.pallas import tpu as pltpu
from jax.experimental.pallas import tpu_sc as plsc


def _build(B, F, V):
    info = plsc.get_sparse_core_info()
    NC, NS, L = info.num_cores, info.num_subcores, info.num_lanes
    NW = NC * NS
    bpw = B // NW
    groups = bpw // L
    FP = 32

    mesh = plsc.VectorSubcoreMesh(core_axis_name="c", subcore_axis_name="s")

    @functools.partial(
        pl.kernel,
        out_type=jax.ShapeDtypeStruct((B,), jnp.float32),
        mesh=mesh,
        compiler_params=pltpu.CompilerParams(needs_layout_passes=False),
        scratch_types=[
            pltpu.VMEM((V,), jnp.float32),
            pltpu.VMEM((bpw * F,), jnp.int32),
            pltpu.VMEM((bpw,), jnp.float32),
            pltpu.VMEM((FP,), jnp.int32),
            pltpu.VMEM((L,), jnp.float32),
            pltpu.SemaphoreType.DMA,
            pltpu.SemaphoreType.DMA,
        ],
    )
    def k(x_hbm, tab_hbm, off_hbm, bias_hbm, out_hbm,
          tab_v, x_v, o_v, off_v, b_v, sem_t, sem_x):
        wid = lax.axis_index("s") * NC + lax.axis_index("c")
        cp_t = pltpu.async_copy(tab_hbm, tab_v, sem_t)
        cp_x = pltpu.async_copy(x_hbm.at[wid], x_v, sem_x)
        pltpu.sync_copy(off_hbm, off_v)
        pltpu.sync_copy(bias_hbm, b_v)

        bias_vec = b_v[...]
        off_vecs = [
            plsc.load_gather(off_v, [jnp.full((L,), f + 1, jnp.int32)])
            for f in range(F)
        ]
        stride_vec = lax.iota(jnp.int32, L) * F

        cp_x.wait()
        cp_t.wait()

        for g in range(groups):
            acc = bias_vec
            gbase = g * L * F
            for f in range(F):
                xv = plsc.load_gather(x_v, [stride_vec + (gbase + f)])
                acc = acc + plsc.load_gather(tab_v, [xv + off_vecs[f]])
            o_v[pl.ds(g * L, L)] = acc
        pltpu.sync_copy(o_v, out_hbm.at[pl.ds(wid * bpw, bpw)])

    return k


def kernel(x, table, offsets, bias):
    B, F = x.shape
    V = table.shape[0]
    NW = 32
    bpw = B // NW
    x_blocks = x.astype(jnp.int32).reshape(NW, bpw * F)
    off_pad = jnp.zeros((32,), jnp.int32).at[1:F + 1].set(offsets.astype(jnp.int32))
    bias_pad = jnp.broadcast_to(bias.astype(jnp.float32), (16,))
    out = _build(B, F, V)(x_blocks, table.reshape(-1), off_pad, bias_pad)
    return out[:, None]
